# Initial kernel scaffold; baseline (speedup 1.0000x reference)
import jax
import jax.numpy as jnp
from jax.experimental import pallas as pl


def kernel(pred, target):
    raise NotImplementedError("write your pallas kernel here")


# speedup vs baseline: 7.1311x; 7.1311x over previous
import jax
import jax.numpy as jnp
from jax.experimental import pallas as pl
from jax.experimental.pallas import tpu as pltpu

VOCAB = 32000
PAD = 0
SMOOTH = 0.1
CONF = 1.0 - SMOOTH
LOW = SMOOTH / (VOCAB - 2)

ROWS = 256
CHUNK = 6400
NCHUNK = VOCAB // CHUNK


def _body(pred_ref, tgt_ref, out_ref, m_ref, se_ref, sp_ref, p0_ref, pt_ref,
          num_ref, den_ref):
    i = pl.program_id(0)
    j = pl.program_id(1)

    chunk = pred_ref[...]
    tgt = tgt_ref[...]

    @pl.when(j == 0)
    def _init_row_block():
        m_ref[...] = jnp.full((ROWS, 1), -jnp.inf, jnp.float32)
        se_ref[...] = jnp.zeros((ROWS, 1), jnp.float32)
        sp_ref[...] = jnp.zeros((ROWS, 1), jnp.float32)
        pt_ref[...] = jnp.zeros((ROWS, 1), jnp.float32)
        p0_ref[...] = chunk[:, 0:1]

    @pl.when((i == 0) & (j == 0))
    def _init_accum():
        num_ref[0, 0] = 0.0
        den_ref[0, 0] = 0.0

    m_old = m_ref[...]
    m_new = jnp.maximum(m_old, jnp.max(chunk, axis=1, keepdims=True))
    alpha = jnp.exp(m_old - m_new)
    cse = jnp.sum(jnp.exp(chunk - m_new), axis=1, keepdims=True)
    se_ref[...] = se_ref[...] * alpha + cse
    sp_ref[...] = sp_ref[...] + jnp.sum(chunk, axis=1, keepdims=True)
    m_ref[...] = m_new

    cols = j * CHUNK + jax.lax.broadcasted_iota(jnp.int32, (ROWS, CHUNK), 1)
    pt_ref[...] += jnp.sum(jnp.where(cols == tgt, chunk, 0.0), axis=1,
                           keepdims=True)

    @pl.when(j == NCHUNK - 1)
    def _finish_row_block():
        L = m_ref[...] + jnp.log(se_ref[...])
        loss = (L - LOW * (sp_ref[...] - p0_ref[...] - pt_ref[...])
                - CONF * pt_ref[...])
        maskf = (tgt != PAD).astype(jnp.float32)
        num_ref[0, 0] += jnp.sum(loss * maskf)
        den_ref[0, 0] += jnp.sum(maskf)

    @pl.when((i == pl.num_programs(0) - 1) & (j == NCHUNK - 1))
    def _emit():
        out_ref[...] = jnp.full(
            (1, 1), num_ref[0, 0] / jnp.maximum(den_ref[0, 0], 1.0),
            jnp.float32)


def kernel(pred, target):
    n = pred.shape[0]
    tgt2d = target.reshape(n, 1)
    out = pl.pallas_call(
        _body,
        grid=(n // ROWS, NCHUNK),
        in_specs=[
            pl.BlockSpec((ROWS, CHUNK), lambda i, j: (i, j)),
            pl.BlockSpec((ROWS, 1), lambda i, j: (i, 0)),
        ],
        out_specs=pl.BlockSpec((1, 1), lambda i, j: (0, 0)),
        out_shape=jax.ShapeDtypeStruct((1, 1), jnp.float32),
        scratch_shapes=[
            pltpu.VMEM((ROWS, 1), jnp.float32),
            pltpu.VMEM((ROWS, 1), jnp.float32),
            pltpu.VMEM((ROWS, 1), jnp.float32),
            pltpu.VMEM((ROWS, 1), jnp.float32),
            pltpu.VMEM((ROWS, 1), jnp.float32),
            pltpu.SMEM((1, 1), jnp.float32),
            pltpu.SMEM((1, 1), jnp.float32),
        ],
    )(pred, tgt2d)
    return out[0, 0]
